# baseline (speedup 1.0000x reference)
import functools

import jax
import jax.numpy as jnp
from jax import lax
from jax.experimental import pallas as pl
from jax.experimental.pallas import tpu as pltpu
from jax.experimental.pallas import tpu_sc as plsc

D0, D1, D2 = 16, 32, 64
NUM_ROWS = 1000000
BATCH = 16384
LANES = 16
NTC = (NUM_ROWS + 127) // 128
TPW = (NTC + 31) // 32
CHUNK = 128
RCHUNKS = BATCH // CHUNK + 1
RROWS = RCHUNKS * CHUNK
NROWS = 32 * RROWS
GW = 160

_info = plsc.get_sparse_core_info()
_NC, _NS = _info.num_cores, _info.num_subcores

_mesh = plsc.VectorSubcoreMesh(core_axis_name="c", subcore_axis_name="s")
_params = pltpu.CompilerParams(needs_layout_passes=False)


@functools.partial(
    pl.kernel,
    mesh=_mesh,
    out_type=(
        jax.ShapeDtypeStruct((NROWS, 128), jnp.float32),
        jax.ShapeDtypeStruct((NROWS,), jnp.int32),
        jax.ShapeDtypeStruct((32, LANES), jnp.int32),
    ),
    scratch_types=[
        pltpu.VMEM((2048,), jnp.int32),
        pltpu.VMEM((BATCH + LANES,), jnp.int32),
        pltpu.VMEM((BATCH + LANES,), jnp.int32),
        pltpu.VMEM((256,), jnp.int32),
        pltpu.VMEM((272,), jnp.int32),
        pltpu.VMEM((4, D0, 128), jnp.float32),
        pltpu.VMEM((4, D1, 128), jnp.float32),
        pltpu.VMEM((4, D2, 128), jnp.float32),
        pltpu.VMEM((CHUNK, 128), jnp.float32),
        pltpu.VMEM((CHUNK,), jnp.int32),
        pltpu.VMEM((16,), jnp.int32),
        pltpu.VMEM((16,), jnp.int32),
        pltpu.VMEM((16,), jnp.int32),
        [pltpu.SemaphoreType.DMA] * 4,
    ],
    compiler_params=_params,
)
def _dedup_gather(idx_hbm, t0_hbm, t1_hbm, t2_hbm,
                  stage_hbm, kstage_hbm, counts_hbm,
                  idxc, hit_i, hit_k, bmap, tcl,
                  rb0, rb1, rb2, rowbuf, rowk, ci, ck, state, sems):
    wid = lax.axis_index("s") * _NC + lax.axis_index("c")
    lo = wid * TPW
    hi = jnp.minimum(lo + TPW, NTC)
    rbase = wid * RROWS
    iota = lax.iota(jnp.int32, LANES)
    ones = jnp.ones((LANES,), jnp.int32)
    neg1 = jnp.full((LANES,), -1, jnp.int32)

    for z in range(16):
        bmap[pl.ds(z * LANES, LANES)] = jnp.zeros((LANES,), jnp.int32)
    for z in range(CHUNK // LANES):
        rowk[pl.ds(z * LANES, LANES)] = neg1

    def scan_chunk(cs, nh0):
        pltpu.sync_copy(idx_hbm.at[pl.ds(cs * 2048, 2048)], idxc)

        def scan_blk(b, nh):
            iv = idxc[pl.ds(b * LANES, LANES)]
            tcv = jax.lax.shift_right_logical(iv, 7)
            m = jnp.logical_and(tcv >= lo, tcv < hi)
            cnt = plsc.all_reduce_population_count(m)[0]
            plsc.store_compressed(hit_i.at[pl.ds(nh, LANES)], iv, mask=m)
            kv = cs * 2048 + b * LANES + iota
            plsc.store_compressed(hit_k.at[pl.ds(nh, LANES)], kv, mask=m)
            plsc.store_scatter(bmap, [tcv - lo], ones, mask=m)
            return nh + cnt

        return lax.fori_loop(0, 2048 // LANES, scan_blk, nh0)

    nh = lax.fori_loop(0, BATCH // 2048, scan_chunk, jnp.int32(0))
    nhb = (nh + LANES - 1) // LANES

    def compact(m16, nm):
        bv = bmap[pl.ds(m16 * LANES, LANES)]
        mk = bv > 0
        plsc.store_compressed(tcl.at[pl.ds(nm, LANES)],
                              iota + m16 * LANES, mask=mk)
        return nm + plsc.all_reduce_population_count(mk)[0]

    nm = lax.fori_loop(0, 16, compact, jnp.int32(0))

    def fetch(slot, tc):
        off = pl.multiple_of(tc * 128, 128)
        pltpu.async_copy(t0_hbm.at[:, pl.ds(off, 128)], rb0.at[slot], sems[slot])
        pltpu.async_copy(t1_hbm.at[:, pl.ds(off, 128)], rb1.at[slot], sems[slot])
        pltpu.async_copy(t2_hbm.at[:, pl.ds(off, 128)], rb2.at[slot], sems[slot])

    def wait(slot):
        pltpu.make_async_copy(t0_hbm.at[:, pl.ds(0, 128)], rb0.at[slot], sems[slot]).wait()
        pltpu.make_async_copy(t1_hbm.at[:, pl.ds(0, 128)], rb1.at[slot], sems[slot]).wait()
        pltpu.make_async_copy(t2_hbm.at[:, pl.ds(0, 128)], rb2.at[slot], sems[slot]).wait()

    def flush_chunk():
        sv = state[pl.ds(0, LANES)]
        nf = sv[1]
        base = pl.multiple_of(rbase + nf * CHUNK, CHUNK)
        pltpu.sync_copy(rowbuf, stage_hbm.at[pl.ds(base, CHUNK)])
        pltpu.sync_copy(rowk, kstage_hbm.at[pl.ds(base, CHUNK)])
        for z in range(CHUNK // LANES):
            rowk[pl.ds(z * LANES, LANES)] = neg1
        state[pl.ds(0, LANES)] = jnp.where(iota == 1, nf + 1, 0)

    def extract_tc(slot, tc_occ):
        def hit_blk(h, _):
            hv = hit_i[pl.ds(h * LANES, LANES)]
            kv = hit_k[pl.ds(h * LANES, LANES)]
            valid = (iota + h * LANES) < nh
            m = jnp.logical_and(
                jax.lax.shift_right_logical(hv, 7) == tc_occ, valid)
            cnt = plsc.all_reduce_population_count(m)[0]

            @pl.when(cnt > 0)
            def _():
                sv0 = state[pl.ds(0, LANES)]

                @pl.when(sv0[0] + cnt > CHUNK)
                def _():
                    flush_chunk()

                sv = state[pl.ds(0, LANES)]
                cp = sv[0]
                plsc.store_compressed(ci.at[pl.ds(0, LANES)], hv, mask=m)
                plsc.store_compressed(ck.at[pl.ds(0, LANES)], kv, mask=m)
                civ = ci[pl.ds(0, LANES)]
                ckv = ck[pl.ds(0, LANES)]
                lanes = jnp.bitwise_and(civ, 127)
                plsc.store_compressed(rowk.at[pl.ds(cp, LANES)], ckv,
                                      mask=iota < cnt)
                slotv = jnp.full((LANES,), slot, jnp.int32)
                for j in range(LANES):
                    @pl.when(j < cnt)
                    def _():
                        lanev = jnp.full((LANES,), lanes[j], jnp.int32)
                        rowv = jnp.full((LANES,), cp + j, jnp.int32)
                        v = plsc.load_gather(rb0, [slotv, iota, lanev])
                        plsc.store_scatter(rowbuf, [rowv, iota], v)
                        for hh in range(D1 // LANES):
                            v = plsc.load_gather(
                                rb1, [slotv, iota + hh * LANES, lanev])
                            plsc.store_scatter(
                                rowbuf, [rowv, iota + D0 + hh * LANES], v)
                        for hh in range(D2 // LANES):
                            v = plsc.load_gather(
                                rb2, [slotv, iota + hh * LANES, lanev])
                            plsc.store_scatter(
                                rowbuf, [rowv, iota + D0 + D1 + hh * LANES], v)
                state[pl.ds(0, LANES)] = jnp.where(iota == 0, cp + cnt, sv)
            return 0

        lax.fori_loop(0, nhb, hit_blk, 0)

    state[pl.ds(0, LANES)] = jnp.zeros((LANES,), jnp.int32)

    def ring(q, _):
        for s in range(4):
            pos = q * 4 + s

            @pl.when(jnp.logical_and(pos >= 4, pos - 4 < nm))
            def _():
                wait(s)

        def exo(s2, _):
            pos = q * 4 + s2

            @pl.when(jnp.logical_and(pos >= 4, pos - 4 < nm))
            def _():
                extract_tc(s2, lo + tcl[pl.ds(pos - 4, LANES)][0])
            return 0

        lax.fori_loop(0, 4, exo, 0)
        for s in range(4):
            pos = q * 4 + s

            @pl.when(pos < nm)
            def _():
                fetch(s, lo + tcl[pl.ds(pos, LANES)][0])
        return 0

    nq = (nm + 7) // 4
    lax.fori_loop(0, nq, ring, 0)

    flush_chunk()
    sv = state[pl.ds(0, LANES)]
    ci[pl.ds(0, LANES)] = jnp.full((LANES,), sv[1], jnp.int32)
    pltpu.sync_copy(ci, counts_hbm.at[wid])


@functools.partial(
    pl.kernel,
    mesh=_mesh,
    out_type=(
        jax.ShapeDtypeStruct((D0, BATCH), jnp.float32),
        jax.ShapeDtypeStruct((D1, BATCH), jnp.float32),
        jax.ShapeDtypeStruct((D2, BATCH), jnp.float32),
    ),
    scratch_types=[
        pltpu.VMEM((16,), jnp.int32),
        pltpu.VMEM((CHUNK,), jnp.int32),
        pltpu.VMEM((4 * GW,), jnp.int32),
        pltpu.VMEM((4 * GW,), jnp.int32),
        pltpu.VMEM((CHUNK, 128), jnp.float32),
        pltpu.VMEM((D0, 128), jnp.float32),
        pltpu.VMEM((D1, 128), jnp.float32),
        pltpu.VMEM((D2, 128), jnp.float32),
        pltpu.SemaphoreType.DMA,
    ],
    compiler_params=_params,
)
def _assemble(stage_hbm, kstage_hbm, counts_hbm, o0_hbm, o1_hbm, o2_hbm,
              cvrow, kch, bldf, kcolf, grows, s0, s1, s2, sem):
    wid = lax.axis_index("s") * _NC + lax.axis_index("c")
    iota = lax.iota(jnp.int32, LANES)
    kbase = wid * 512

    carry0 = (jnp.int32(0), jnp.int32(0), jnp.int32(0), jnp.int32(0))

    def scan_w(carry, w):
        pltpu.sync_copy(counts_hbm.at[w], cvrow)
        nf = cvrow[pl.ds(0, LANES)][0]

        def scan_chunk(ch, carry):
            cbase = pl.multiple_of(w * RROWS + ch * CHUNK, CHUNK)
            pltpu.sync_copy(kstage_hbm.at[pl.ds(cbase, CHUNK)], kch)

            def scan_step(st, carry):
                kv = kch[pl.ds(st * LANES, LANES)]
                rowpos = cbase + st * LANES + iota
                out = []
                for g in range(4):
                    cg = carry[g]
                    glo = kbase + g * 128
                    mg = jnp.logical_and(kv >= glo, kv < glo + 128)
                    plsc.store_compressed(
                        bldf.at[pl.ds(g * GW + cg, LANES)], rowpos, mask=mg)
                    plsc.store_compressed(
                        kcolf.at[pl.ds(g * GW + cg, LANES)], kv, mask=mg)
                    out.append(cg + plsc.all_reduce_population_count(mg)[0])
                return tuple(out)

            return lax.fori_loop(0, CHUNK // LANES, scan_step, carry)

        return lax.fori_loop(0, nf, scan_chunk, carry)

    carry = carry0
    for w in range(32):
        carry = scan_w(carry, w)

    for g in range(4):
        pltpu.async_copy(
            stage_hbm.at[bldf.at[pl.ds(g * GW, 128)]], grows, sem).wait()
        def trans_blk(kb, _):
            kcv = kcolf[pl.ds(g * GW + kb * LANES, LANES)]
            colv = jnp.bitwise_and(kcv, 127)
            for j in range(LANES):
                colsplat = jnp.full((LANES,), colv[j], jnp.int32)
                rowsplat = jnp.full((LANES,), kb * LANES + j, jnp.int32)
                v = plsc.load_gather(grows, [rowsplat, iota])
                plsc.store_scatter(s0, [iota, colsplat], v)
                for hh in range(D1 // LANES):
                    v = plsc.load_gather(grows, [rowsplat, iota + D0 + hh * LANES])
                    plsc.store_scatter(s1, [iota + hh * LANES, colsplat], v)
                for hh in range(D2 // LANES):
                    v = plsc.load_gather(
                        grows, [rowsplat, iota + D0 + D1 + hh * LANES])
                    plsc.store_scatter(s2, [iota + hh * LANES, colsplat], v)
            return 0

        lax.fori_loop(0, CHUNK // LANES, trans_blk, 0)
        off = pl.multiple_of(kbase + g * 128, 128)
        pltpu.sync_copy(s0, o0_hbm.at[:, pl.ds(off, 128)])
        pltpu.sync_copy(s1, o1_hbm.at[:, pl.ds(off, 128)])
        pltpu.sync_copy(s2, o2_hbm.at[:, pl.ds(off, 128)])


def kernel(task_id, table0, table1, table2):
    idx = task_id.astype(jnp.int32)
    stage, kstage, counts = _dedup_gather(
        idx, table0.T, table1.T, table2.T)
    o0t, o1t, o2t = _assemble(stage, kstage, counts)
    return o0t.T, o1t.T, o2t.T


# speedup vs baseline: 2.0173x; 2.0173x over previous
import functools

import jax
import jax.numpy as jnp
from jax import lax
from jax.experimental import pallas as pl
from jax.experimental.pallas import tpu as pltpu
from jax.experimental.pallas import tpu_sc as plsc

D0, D1, D2 = 16, 32, 64
NUM_ROWS = 1000000
BATCH = 16384
LANES = 16

_info = plsc.get_sparse_core_info()
_NC, _NS = _info.num_cores, _info.num_subcores
_NW = _NC * _NS
_BPW = BATCH // _NW
_NBUF = 4
_GRP = 128

_mesh = plsc.VectorSubcoreMesh(core_axis_name="c", subcore_axis_name="s")


@functools.partial(
    pl.kernel,
    mesh=_mesh,
    out_type=(
        jax.ShapeDtypeStruct((D0, BATCH), jnp.float32),
        jax.ShapeDtypeStruct((D1, BATCH), jnp.float32),
        jax.ShapeDtypeStruct((D2, BATCH), jnp.float32),
    ),
    scratch_types=[
        pltpu.VMEM((_BPW,), jnp.int32),
        pltpu.VMEM((_NBUF, D0, 128), jnp.float32),
        pltpu.VMEM((_NBUF, D1, 128), jnp.float32),
        pltpu.VMEM((_NBUF, D2, 128), jnp.float32),
        pltpu.VMEM((2, D0, _GRP), jnp.float32),
        pltpu.VMEM((2, D1, _GRP), jnp.float32),
        pltpu.VMEM((2, D2, _GRP), jnp.float32),
        [pltpu.SemaphoreType.DMA] * _NBUF,
        [pltpu.SemaphoreType.DMA] * 2,
    ],
    compiler_params=pltpu.CompilerParams(needs_layout_passes=False),
)
def _emb_lookup(idx_hbm, t0_hbm, t1_hbm, t2_hbm, o0_hbm, o1_hbm, o2_hbm,
                idx_v, rb0, rb1, rb2, s0, s1, s2, sems, fsems):
    wid = lax.axis_index("s") * _NC + lax.axis_index("c")
    base = wid * _BPW
    pltpu.sync_copy(idx_hbm.at[pl.ds(base, _BPW)], idx_v)
    iota = lax.iota(jnp.int32, LANES)

    def fetch(slot, col_off):
        off = pl.multiple_of(col_off, 128)
        pltpu.async_copy(t0_hbm.at[:, pl.ds(off, 128)], rb0.at[slot], sems[slot])
        pltpu.async_copy(t1_hbm.at[:, pl.ds(off, 128)], rb1.at[slot], sems[slot])
        pltpu.async_copy(t2_hbm.at[:, pl.ds(off, 128)], rb2.at[slot], sems[slot])

    def wait(slot):
        pltpu.make_async_copy(t0_hbm.at[:, pl.ds(0, 128)], rb0.at[slot], sems[slot]).wait()
        pltpu.make_async_copy(t1_hbm.at[:, pl.ds(0, 128)], rb1.at[slot], sems[slot]).wait()
        pltpu.make_async_copy(t2_hbm.at[:, pl.ds(0, 128)], rb2.at[slot], sems[slot]).wait()

    def extract(slot, lane, kcol, par):
        lanev = jnp.full((LANES,), lane, jnp.int32)
        kv = jnp.full((LANES,), kcol, jnp.int32)
        pv = jnp.full((LANES,), par, jnp.int32)
        v = plsc.load_gather(rb0.at[slot], [iota, lanev])
        plsc.store_scatter(s0, [pv, iota, kv], v)
        for h in range(D1 // LANES):
            v = plsc.load_gather(rb1.at[slot], [iota + h * LANES, lanev])
            plsc.store_scatter(s1, [pv, iota + h * LANES, kv], v)
        for h in range(D2 // LANES):
            v = plsc.load_gather(rb2.at[slot], [iota + h * LANES, lanev])
            plsc.store_scatter(s2, [pv, iota + h * LANES, kv], v)

    def flush_async(grp_off, p):
        off = pl.multiple_of(base + grp_off, 128)
        pltpu.async_copy(s0.at[p], o0_hbm.at[:, pl.ds(off, _GRP)], fsems[p])
        pltpu.async_copy(s1.at[p], o1_hbm.at[:, pl.ds(off, _GRP)], fsems[p])
        pltpu.async_copy(s2.at[p], o2_hbm.at[:, pl.ds(off, _GRP)], fsems[p])

    def flush_wait(p):
        pltpu.make_async_copy(s0.at[p], o0_hbm.at[:, pl.ds(0, _GRP)], fsems[p]).wait()
        pltpu.make_async_copy(s1.at[p], o1_hbm.at[:, pl.ds(0, _GRP)], fsems[p]).wait()
        pltpu.make_async_copy(s2.at[p], o2_hbm.at[:, pl.ds(0, _GRP)], fsems[p]).wait()

    def body(blk, carry):
        cv_prev, lv_prev = carry
        kk0 = blk * LANES
        iv = idx_v[pl.ds(kk0, LANES)]
        lv = jnp.bitwise_and(iv, 127)
        cv = iv - lv
        for j in range(LANES):
            kk = kk0 + j
            if j < _NBUF:
                kko = kk - _NBUF
                @pl.when(blk > 0)
                def _():
                    wait(j)
                    extract(j, lv_prev[j + LANES - _NBUF],
                            kko % _GRP, (kko // _GRP) % 2)
                if j == _NBUF - 1:
                    @pl.when(jnp.logical_and(blk > 0, blk % 16 == 8))
                    def _():
                        flush_async((blk - 8) * LANES, 0)

                    @pl.when(jnp.logical_and(blk > 8, blk % 16 == 8))
                    def _():
                        flush_wait(1)

                    @pl.when(jnp.logical_and(blk > 0, blk % 16 == 0))
                    def _():
                        flush_async((blk - 8) * LANES, 1)
                        flush_wait(0)
                fetch(j, cv[j])
            else:
                kko = kk - _NBUF
                wait(j % _NBUF)
                extract(j % _NBUF, lv[j - _NBUF],
                        kko % _GRP, (kko // _GRP) % 2)
                fetch(j % _NBUF, cv[j])
        return cv, lv

    zero = jnp.zeros((LANES,), jnp.int32)
    cv_last, lv_last = lax.fori_loop(0, _BPW // LANES, body, (zero, zero))

    for j in range(_NBUF):
        kk = _BPW - _NBUF + j
        wait(j % _NBUF)
        extract(j % _NBUF, lv_last[j + LANES - _NBUF], kk % _GRP,
                ((_BPW - _GRP) // _GRP) % 2)
    flush_wait(0)
    flush_async(_BPW - _GRP, 1)
    flush_wait(1)


def kernel(task_id, table0, table1, table2):
    o0t, o1t, o2t = _emb_lookup(
        task_id.astype(jnp.int32), table0.T, table1.T, table2.T
    )
    return o0t.T, o1t.T, o2t.T


# speedup vs baseline: 2.2482x; 1.1144x over previous
import functools

import jax
import jax.numpy as jnp
from jax import lax
from jax.experimental import pallas as pl
from jax.experimental.pallas import tpu as pltpu
from jax.experimental.pallas import tpu_sc as plsc

D0, D1, D2 = 16, 32, 64
NUM_ROWS = 1000000
BATCH = 16384
LANES = 16

_info = plsc.get_sparse_core_info()
_NC, _NS = _info.num_cores, _info.num_subcores
_NW = _NC * _NS
_BPW = BATCH // _NW
_NBUF = 8
_GRP = 128

_mesh = plsc.VectorSubcoreMesh(core_axis_name="c", subcore_axis_name="s")


@functools.partial(
    pl.kernel,
    mesh=_mesh,
    out_type=(
        jax.ShapeDtypeStruct((D0, BATCH), jnp.float32),
        jax.ShapeDtypeStruct((D1, BATCH), jnp.float32),
        jax.ShapeDtypeStruct((D2, BATCH), jnp.float32),
    ),
    scratch_types=[
        pltpu.VMEM((_BPW,), jnp.int32),
        pltpu.VMEM((_NBUF, D0, 128), jnp.float32),
        pltpu.VMEM((_NBUF, D1, 128), jnp.float32),
        pltpu.VMEM((_NBUF, D2, 128), jnp.float32),
        pltpu.VMEM((D0, _GRP), jnp.float32),
        pltpu.VMEM((D1, _GRP), jnp.float32),
        pltpu.VMEM((D2, _GRP), jnp.float32),
        [pltpu.SemaphoreType.DMA] * _NBUF,
    ],
    compiler_params=pltpu.CompilerParams(needs_layout_passes=False),
)
def _emb_lookup(idx_hbm, t0_hbm, t1_hbm, t2_hbm, o0_hbm, o1_hbm, o2_hbm,
                idx_v, rb0, rb1, rb2, s0, s1, s2, sems):
    wid = lax.axis_index("s") * _NC + lax.axis_index("c")
    base = wid * _BPW
    pltpu.sync_copy(idx_hbm.at[pl.ds(base, _BPW)], idx_v)
    iota = lax.iota(jnp.int32, LANES)

    def fetch(slot, col_off):
        off = pl.multiple_of(col_off, 128)
        pltpu.async_copy(t0_hbm.at[:, pl.ds(off, 128)], rb0.at[slot], sems[slot])
        pltpu.async_copy(t1_hbm.at[:, pl.ds(off, 128)], rb1.at[slot], sems[slot])
        pltpu.async_copy(t2_hbm.at[:, pl.ds(off, 128)], rb2.at[slot], sems[slot])

    def wait(slot):
        pltpu.make_async_copy(t0_hbm.at[:, pl.ds(0, 128)], rb0.at[slot], sems[slot]).wait()
        pltpu.make_async_copy(t1_hbm.at[:, pl.ds(0, 128)], rb1.at[slot], sems[slot]).wait()
        pltpu.make_async_copy(t2_hbm.at[:, pl.ds(0, 128)], rb2.at[slot], sems[slot]).wait()

    def extract(slot, lane, kcol):
        lanev = jnp.full((LANES,), lane, jnp.int32)
        kv = jnp.full((LANES,), kcol, jnp.int32)
        v = plsc.load_gather(rb0.at[slot], [iota, lanev])
        plsc.store_scatter(s0, [iota, kv], v)
        for h in range(D1 // LANES):
            v = plsc.load_gather(rb1.at[slot], [iota + h * LANES, lanev])
            plsc.store_scatter(s1, [iota + h * LANES, kv], v)
        for h in range(D2 // LANES):
            v = plsc.load_gather(rb2.at[slot], [iota + h * LANES, lanev])
            plsc.store_scatter(s2, [iota + h * LANES, kv], v)

    def flush(grp_off):
        off = pl.multiple_of(base + grp_off, 128)
        pltpu.sync_copy(s0, o0_hbm.at[:, pl.ds(off, _GRP)])
        pltpu.sync_copy(s1, o1_hbm.at[:, pl.ds(off, _GRP)])
        pltpu.sync_copy(s2, o2_hbm.at[:, pl.ds(off, _GRP)])

    def body(blk, carry):
        cv_prev, lv_prev = carry
        kk0 = blk * LANES
        iv = idx_v[pl.ds(kk0, LANES)]
        lv = jnp.bitwise_and(iv, 127)
        cv = iv - lv
        for j in range(LANES):
            kk = kk0 + j
            if j < _NBUF:
                @pl.when(blk > 0)
                def _():
                    wait(j)
                    extract(j, lv_prev[j + LANES - _NBUF],
                            (kk - _NBUF) % _GRP)
                if j == _NBUF - 1:
                    @pl.when(jnp.logical_and(blk > 0, blk % 8 == 0))
                    def _():
                        flush((blk - 8) * LANES)
                fetch(j, cv[j])
            else:
                wait(j % _NBUF)
                extract(j % _NBUF, lv[j - _NBUF], (kk - _NBUF) % _GRP)
                fetch(j % _NBUF, cv[j])
        return cv, lv

    zero = jnp.zeros((LANES,), jnp.int32)
    cv_last, lv_last = lax.fori_loop(0, _BPW // LANES, body, (zero, zero))

    for j in range(_NBUF):
        kk = _BPW - _NBUF + j
        wait(j % _NBUF)
        extract(j % _NBUF, lv_last[j + LANES - _NBUF], kk % _GRP)
    flush(_BPW - _GRP)


def kernel(task_id, table0, table1, table2):
    o0t, o1t, o2t = _emb_lookup(
        task_id.astype(jnp.int32), table0.T, table1.T, table2.T
    )
    return o0t.T, o1t.T, o2t.T
